# Initial kernel scaffold; baseline (speedup 1.0000x reference)
import jax
import jax.numpy as jnp
from jax.experimental import pallas as pl


def kernel(x, W1, b1, W2, b2, W3, b3):
    raise NotImplementedError("write your pallas kernel here")


# speedup vs baseline: 5.0714x; 5.0714x over previous
import functools

import jax
import jax.numpy as jnp
from jax import lax
from jax.experimental import pallas as pl
from jax.experimental.pallas import tpu as pltpu
from jax.experimental.pallas import tpu_sc as plsc

N = 10000
D = 128
HID = 128
K = 16

ROWS = 200
NEG = -1e30
BIGI = 2 ** 30

SC_CORES = 2
SC_SUBCORES = 16
SC_WORKERS = SC_CORES * SC_SUBCORES
NPAD = 10240
ROWS_PER_W = NPAD // SC_WORKERS
CHUNK = 8
CHUNKS_PER_W = ROWS_PER_W // CHUNK


def _knn_body(x_ref, xf_ref, idx_ref):
    xt = x_ref[...]
    xf = xf_ref[...]
    sq = jnp.sum(xf * xf, axis=1)
    g = lax.dot_general(xt.astype(jnp.bfloat16), xf.astype(jnp.bfloat16),
                        (((1,), (1,)), ((), ())),
                        preferred_element_type=jnp.float32)
    s = 2.0 * g - sq[None, :]
    pos = lax.broadcasted_iota(jnp.int32, (ROWS, N), 1)
    cols = []
    for _ in range(K):
        m = jnp.max(s, axis=1, keepdims=True)
        p = jnp.min(jnp.where(s == m, pos, BIGI), axis=1, keepdims=True)
        sel = pos == p
        cols.append(p)
        s = jnp.where(sel, NEG, s)
    idx_ref[...] = jnp.concatenate(cols, axis=1)


def _knn(h):
    return pl.pallas_call(
        _knn_body,
        grid=(N // ROWS,),
        in_specs=[
            pl.BlockSpec((ROWS, D), lambda i: (i, 0)),
            pl.BlockSpec((N, D), lambda i: (0, 0)),
        ],
        out_specs=pl.BlockSpec((ROWS, K), lambda i: (i, 0)),
        out_shape=jax.ShapeDtypeStruct((N, K), jnp.int32),
        compiler_params=pltpu.CompilerParams(
            dimension_semantics=("arbitrary",)),
    )(h, h)


def _gather_body(x_hbm, idx_hbm, out_hbm, idx_v, rows_v, sem):
    wid = lax.axis_index("s") * SC_CORES + lax.axis_index("c")
    base = wid * ROWS_PER_W

    def chunk_body(ci, carry):
        eb = (base + ci * CHUNK) * K
        pltpu.sync_copy(idx_hbm.at[pl.ds(eb, CHUNK * K)], idx_v)
        pltpu.async_copy(x_hbm.at[idx_v], rows_v, sem).wait()
        pltpu.sync_copy(rows_v, out_hbm.at[pl.ds(eb, CHUNK * K)])
        return carry

    lax.fori_loop(0, CHUNKS_PER_W, chunk_body, 0)


@functools.cache
def _gather_kernel():
    return pl.kernel(
        _gather_body,
        out_type=jax.ShapeDtypeStruct((NPAD * K, D), jnp.float32),
        mesh=plsc.VectorSubcoreMesh(core_axis_name="c", subcore_axis_name="s"),
        scratch_types=[
            pltpu.VMEM((CHUNK * K,), jnp.int32),
            pltpu.VMEM((CHUNK * K, D), jnp.float32),
            pltpu.SemaphoreType.DMA,
        ],
    )


def _edge_mlp_body(x_ref, xj_ref, wt_ref, b_ref, out_ref):
    xt = x_ref[...]
    xj = xj_ref[...]
    xi = jnp.reshape(
        jnp.broadcast_to(xt[:, None, :], (ROWS, K, D)), (ROWS * K, D))
    feat = jnp.concatenate([xi, xj - xi], axis=1)
    msg = lax.dot_general(feat.astype(jnp.bfloat16),
                          wt_ref[...].astype(jnp.bfloat16),
                          (((1,), (0,)), ((), ())),
                          preferred_element_type=jnp.float32)
    msg = msg + b_ref[...]
    msg = jnp.where(msg >= 0.0, msg, 0.2 * msg)
    out_ref[...] = jnp.max(jnp.reshape(msg, (ROWS, K, HID)), axis=1)


def _edge_mlp(h, xj, wt, b2d):
    return pl.pallas_call(
        _edge_mlp_body,
        grid=(N // ROWS,),
        in_specs=[
            pl.BlockSpec((ROWS, D), lambda i: (i, 0)),
            pl.BlockSpec((ROWS * K, D), lambda i: (i, 0)),
            pl.BlockSpec((2 * D, HID), lambda i: (0, 0)),
            pl.BlockSpec((1, HID), lambda i: (0, 0)),
        ],
        out_specs=pl.BlockSpec((ROWS, HID), lambda i: (i, 0)),
        out_shape=jax.ShapeDtypeStruct((N, HID), jnp.float32),
        compiler_params=pltpu.CompilerParams(
            dimension_semantics=("arbitrary",)),
    )(h, xj, wt, b2d)


def _edge_conv(h, W, b):
    idx = _knn(h)
    idx_flat = jnp.reshape(idx, (N * K,))
    idx_pad = jnp.pad(idx_flat, (0, (NPAD - N) * K))
    xj = _gather_kernel()(h, idx_pad)
    wt = jnp.transpose(W)
    b2d = jnp.reshape(b, (1, HID))
    return _edge_mlp(h, xj, wt, b2d)


def kernel(x, W1, b1, W2, b2, W3, b3):
    h = _edge_conv(x, W1, b1)
    h = _edge_conv(h, W2, b2)
    h = _edge_conv(h, W3, b3)
    return h


# speedup vs baseline: 5.5606x; 1.0965x over previous
import functools

import jax
import jax.numpy as jnp
from jax import lax
from jax.experimental import pallas as pl
from jax.experimental.pallas import tpu as pltpu
from jax.experimental.pallas import tpu_sc as plsc

N = 10000
D = 128
HID = 128
K = 16

ROWS = 200
NEG = -1e30
BIGI = 2 ** 30

SC_CORES = 2
SC_SUBCORES = 16
SC_WORKERS = SC_CORES * SC_SUBCORES
NPAD = 10240
ROWS_PER_W = NPAD // SC_WORKERS
CHUNK = 8
CHUNKS_PER_W = ROWS_PER_W // CHUNK


def _knn_body(x_ref, xf_ref, idx_ref):
    xt = x_ref[...]
    xf = xf_ref[...]
    sq = jnp.sum(xf * xf, axis=1)
    g = lax.dot_general(xt.astype(jnp.bfloat16), xf.astype(jnp.bfloat16),
                        (((1,), (1,)), ((), ())),
                        preferred_element_type=jnp.float32)
    s = 2.0 * g - sq[None, :]
    pos = lax.broadcasted_iota(jnp.int32, (ROWS, N), 1)
    cols = []
    for t in range(K):
        m = jnp.max(s, axis=1, keepdims=True)
        eq = s == m
        p = jnp.min(jnp.where(eq, pos, BIGI), axis=1, keepdims=True)
        cols.append(p)
        if t + 1 < K:
            s = jnp.where(eq, NEG, s)
    idx_ref[...] = jnp.concatenate(cols, axis=1)


def _knn(h):
    return pl.pallas_call(
        _knn_body,
        grid=(N // ROWS,),
        in_specs=[
            pl.BlockSpec((ROWS, D), lambda i: (i, 0)),
            pl.BlockSpec((N, D), lambda i: (0, 0)),
        ],
        out_specs=pl.BlockSpec((ROWS, K), lambda i: (i, 0)),
        out_shape=jax.ShapeDtypeStruct((N, K), jnp.int32),
        compiler_params=pltpu.CompilerParams(
            dimension_semantics=("parallel",)),
    )(h, h)


def _gather_body(x_hbm, idx_hbm, out_hbm, idx_v, rows_v, sem):
    wid = lax.axis_index("s") * SC_CORES + lax.axis_index("c")
    base = wid * ROWS_PER_W

    def chunk_body(ci, carry):
        eb = (base + ci * CHUNK) * K
        pltpu.sync_copy(idx_hbm.at[pl.ds(eb, CHUNK * K)], idx_v)
        pltpu.async_copy(x_hbm.at[idx_v], rows_v, sem).wait()
        pltpu.sync_copy(rows_v, out_hbm.at[pl.ds(eb, CHUNK * K)])
        return carry

    lax.fori_loop(0, CHUNKS_PER_W, chunk_body, 0)


@functools.cache
def _gather_kernel():
    return pl.kernel(
        _gather_body,
        out_type=jax.ShapeDtypeStruct((NPAD * K, D), jnp.float32),
        mesh=plsc.VectorSubcoreMesh(core_axis_name="c", subcore_axis_name="s"),
        scratch_types=[
            pltpu.VMEM((CHUNK * K,), jnp.int32),
            pltpu.VMEM((CHUNK * K, D), jnp.float32),
            pltpu.SemaphoreType.DMA,
        ],
    )


def _edge_mlp_body(x_ref, xj_ref, wt_ref, b_ref, out_ref):
    xt = x_ref[...]
    xj = xj_ref[...]
    xi = jnp.reshape(
        jnp.broadcast_to(xt[:, None, :], (ROWS, K, D)), (ROWS * K, D))
    feat = jnp.concatenate([xi, xj - xi], axis=1)
    msg = lax.dot_general(feat.astype(jnp.bfloat16),
                          wt_ref[...].astype(jnp.bfloat16),
                          (((1,), (0,)), ((), ())),
                          preferred_element_type=jnp.float32)
    msg = msg + b_ref[...]
    msg = jnp.where(msg >= 0.0, msg, 0.2 * msg)
    out_ref[...] = jnp.max(jnp.reshape(msg, (ROWS, K, HID)), axis=1)


def _edge_mlp(h, xj, wt, b2d):
    return pl.pallas_call(
        _edge_mlp_body,
        grid=(N // ROWS,),
        in_specs=[
            pl.BlockSpec((ROWS, D), lambda i: (i, 0)),
            pl.BlockSpec((ROWS * K, D), lambda i: (i, 0)),
            pl.BlockSpec((2 * D, HID), lambda i: (0, 0)),
            pl.BlockSpec((1, HID), lambda i: (0, 0)),
        ],
        out_specs=pl.BlockSpec((ROWS, HID), lambda i: (i, 0)),
        out_shape=jax.ShapeDtypeStruct((N, HID), jnp.float32),
        compiler_params=pltpu.CompilerParams(
            dimension_semantics=("parallel",)),
    )(h, xj, wt, b2d)


def _edge_conv(h, W, b):
    idx = _knn(h)
    idx_flat = jnp.reshape(idx, (N * K,))
    idx_pad = jnp.pad(idx_flat, (0, (NPAD - N) * K))
    xj = _gather_kernel()(h, idx_pad)
    wt = jnp.transpose(W)
    b2d = jnp.reshape(b, (1, HID))
    return _edge_mlp(h, xj, wt, b2d)


def kernel(x, W1, b1, W2, b2, W3, b3):
    h = _edge_conv(x, W1, b1)
    h = _edge_conv(h, W2, b2)
    h = _edge_conv(h, W3, b3)
    return h


# speedup vs baseline: 5.6861x; 1.0226x over previous
import functools

import jax
import jax.numpy as jnp
from jax import lax
from jax.experimental import pallas as pl
from jax.experimental.pallas import tpu as pltpu
from jax.experimental.pallas import tpu_sc as plsc

N = 10000
D = 128
HID = 128
K = 16

ROWS = 200
NEG = -1e30
BIGI = 2 ** 30

SC_CORES = 2
SC_SUBCORES = 16
SC_WORKERS = SC_CORES * SC_SUBCORES
NPAD = 10240
ROWS_PER_W = NPAD // SC_WORKERS
CHUNK = 8
CHUNKS_PER_W = ROWS_PER_W // CHUNK


def _knn_body(x_ref, xf_ref, idx_ref):
    xt = x_ref[...]
    xf = xf_ref[...]
    sq = jnp.sum(xf * xf, axis=1)
    g = lax.dot_general(xt.astype(jnp.bfloat16), xf.astype(jnp.bfloat16),
                        (((1,), (1,)), ((), ())),
                        preferred_element_type=jnp.float32)
    s = 2.0 * g - sq[None, :]
    pos = lax.broadcasted_iota(jnp.int32, (ROWS, N), 1)
    cols = []
    for t in range(K):
        m = jnp.max(s, axis=1, keepdims=True)
        eq = s == m
        p = jnp.min(jnp.where(eq, pos, BIGI), axis=1, keepdims=True)
        cols.append(p)
        if t + 1 < K:
            s = jnp.where(eq, NEG, s)
    idx_ref[...] = jnp.concatenate(cols, axis=1)


def _knn(h):
    return pl.pallas_call(
        _knn_body,
        grid=(N // ROWS,),
        in_specs=[
            pl.BlockSpec((ROWS, D), lambda i: (i, 0)),
            pl.BlockSpec((N, D), lambda i: (0, 0)),
        ],
        out_specs=pl.BlockSpec((ROWS, K), lambda i: (i, 0)),
        out_shape=jax.ShapeDtypeStruct((N, K), jnp.int32),
        compiler_params=pltpu.CompilerParams(
            dimension_semantics=("parallel",)),
    )(h, h)


NBUF = 4


def _gather_body(x_hbm, idx_hbm, out_hbm, *scratch):
    idx_bufs = scratch[0:NBUF]
    row_bufs = scratch[NBUF:2 * NBUF]
    sems = scratch[2 * NBUF:3 * NBUF]
    wid = lax.axis_index("s") * SC_CORES + lax.axis_index("c")
    base = wid * ROWS_PER_W

    def start(ci, b):
        eb = (base + ci * CHUNK) * K
        pltpu.sync_copy(idx_hbm.at[pl.ds(eb, CHUNK * K)], idx_bufs[b])
        pltpu.async_copy(x_hbm.at[idx_bufs[b]], row_bufs[b], sems[b])

    def finish(ci, b):
        pltpu.make_async_copy(x_hbm.at[idx_bufs[b]], row_bufs[b], sems[b]).wait()
        eb = (base + ci * CHUNK) * K
        pltpu.sync_copy(row_bufs[b], out_hbm.at[pl.ds(eb, CHUNK * K)])

    for b in range(NBUF):
        start(b, b)

    def pipe_body(pi, carry):
        for b in range(NBUF):
            ci = pi * NBUF + b
            finish(ci, b)
            start(ci + NBUF, b)
        return carry

    lax.fori_loop(0, CHUNKS_PER_W // NBUF - 1, pipe_body, 0)
    for b in range(NBUF):
        finish(CHUNKS_PER_W - NBUF + b, b)


@functools.cache
def _gather_kernel():
    return pl.kernel(
        _gather_body,
        out_type=jax.ShapeDtypeStruct((NPAD * K, D), jnp.float32),
        mesh=plsc.VectorSubcoreMesh(core_axis_name="c", subcore_axis_name="s"),
        scratch_types=(
            [pltpu.VMEM((CHUNK * K,), jnp.int32) for _ in range(NBUF)]
            + [pltpu.VMEM((CHUNK * K, D), jnp.float32) for _ in range(NBUF)]
            + [pltpu.SemaphoreType.DMA for _ in range(NBUF)]
        ),
    )


def _edge_mlp_body(x_ref, xj_ref, wt_ref, b_ref, out_ref):
    xt = x_ref[...]
    xj = xj_ref[...]
    xi = jnp.reshape(
        jnp.broadcast_to(xt[:, None, :], (ROWS, K, D)), (ROWS * K, D))
    feat = jnp.concatenate([xi, xj - xi], axis=1)
    msg = lax.dot_general(feat.astype(jnp.bfloat16),
                          wt_ref[...].astype(jnp.bfloat16),
                          (((1,), (0,)), ((), ())),
                          preferred_element_type=jnp.float32)
    msg = msg + b_ref[...]
    msg = jnp.where(msg >= 0.0, msg, 0.2 * msg)
    out_ref[...] = jnp.max(jnp.reshape(msg, (ROWS, K, HID)), axis=1)


def _edge_mlp(h, xj, wt, b2d):
    return pl.pallas_call(
        _edge_mlp_body,
        grid=(N // ROWS,),
        in_specs=[
            pl.BlockSpec((ROWS, D), lambda i: (i, 0)),
            pl.BlockSpec((ROWS * K, D), lambda i: (i, 0)),
            pl.BlockSpec((2 * D, HID), lambda i: (0, 0)),
            pl.BlockSpec((1, HID), lambda i: (0, 0)),
        ],
        out_specs=pl.BlockSpec((ROWS, HID), lambda i: (i, 0)),
        out_shape=jax.ShapeDtypeStruct((N, HID), jnp.float32),
        compiler_params=pltpu.CompilerParams(
            dimension_semantics=("parallel",)),
    )(h, xj, wt, b2d)


def _edge_conv(h, W, b):
    idx = _knn(h)
    idx_flat = jnp.reshape(idx, (N * K,))
    idx_pad = jnp.pad(idx_flat, (0, (NPAD - N) * K))
    xj = _gather_kernel()(h, idx_pad)
    wt = jnp.transpose(W)
    b2d = jnp.reshape(b, (1, HID))
    return _edge_mlp(h, xj, wt, b2d)


def kernel(x, W1, b1, W2, b2, W3, b3):
    h = _edge_conv(x, W1, b1)
    h = _edge_conv(h, W2, b2)
    h = _edge_conv(h, W3, b3)
    return h
